# baseline (speedup 1.0000x reference)
import jax
import jax.numpy as jnp
from jax import lax
from jax.experimental import pallas as pl
from jax.experimental.pallas import tpu as pltpu
from jax.experimental.pallas import tpu_sc as plsc

N_NODES = 10000
N_EDGES = 320000
D_NODE = 128
D_EDGE = 16

NC = 2
NS = 16
NW = NC * NS
GRP_EDGES = 512
GROUPS = 20
EDGES_PER_TILE = GRP_EDGES * GROUPS
E_PAD = NW * EDGES_PER_TILE


def _pq_body(ns_ref, wa_ref, wb_ref, t_ref):
    half = pl.num_programs(0) // 2
    t = pl.program_id(0)
    w = jnp.where(t < half, wa_ref[...], wb_ref[...])
    t_ref[...] = jnp.dot(ns_ref[...], w, preferred_element_type=jnp.float32)


def _precompute_table(ns, wa, wb):
    br = 2000
    nb = N_NODES // br
    return pl.pallas_call(
        _pq_body,
        grid=(2 * nb,),
        in_specs=[
            pl.BlockSpec((br, D_NODE), lambda t: (t % (N_NODES // 2000), 0)),
            pl.BlockSpec((D_NODE, D_EDGE), lambda t: (0, 0)),
            pl.BlockSpec((D_NODE, D_EDGE), lambda t: (0, 0)),
        ],
        out_specs=pl.BlockSpec((br, D_EDGE), lambda t: (t, 0)),
        out_shape=jax.ShapeDtypeStruct((2 * N_NODES, D_EDGE), jnp.float32),
    )(ns, wa, wb)


def _gather_body(t_hbm, cidx_hbm, st_hbm,
                 idx_v, a0, a1, b0, b1, c0, c1, gsem, wsem):
    wid = lax.axis_index("s") * NC + lax.axis_index("c")
    pltpu.sync_copy(cidx_hbm.at[wid], idx_v)
    base = wid * EDGES_PER_TILE
    ab = ((a0, b0), (a1, b1))
    cb = (c0, c1)
    iota16 = lax.iota(jnp.int32, 16)

    def g_pairs(g, par):
        sl = pl.ds(g * GRP_EDGES, GRP_EDGES)
        return ((t_hbm.at[idx_v.at[0, sl]], ab[par][0], gsem),
                (t_hbm.at[idx_v.at[1, sl]], ab[par][1], gsem))

    def w_pair(g, par):
        e0 = base + g * GRP_EDGES
        return ((cb[par], st_hbm.at[:, pl.ds(e0, GRP_EDGES)], wsem),)

    def fire(pairs):
        for s, d, sem in pairs:
            pltpu.async_copy(s, d, sem)

    def drain(pairs):
        for s, d, sem in pairs:
            pltpu.make_async_copy(s, d, sem).wait()

    fire(g_pairs(0, 0))
    for g in range(GROUPS):
        par = g & 1
        if g + 1 < GROUPS:
            fire(g_pairs(g + 1, 1 - par))
        drain(g_pairs(g, par))
        if g >= 2:
            drain(w_pair(g - 2, par))
        a, b = ab[par]
        c = cb[par]

        @pl.loop(0, GRP_EDGES // 16)
        def _addt(eb):
            rows = eb * 16 + iota16
            for j in range(D_EDGE):
                col = jnp.full((16,), j, jnp.int32)
                va = plsc.load_gather(a, [rows, col])
                vb = plsc.load_gather(b, [rows, col])
                c[j, pl.ds(eb * 16, 16)] = va + vb

        fire(w_pair(g, par))
    drain(w_pair(GROUPS - 2, (GROUPS - 2) & 1))
    drain(w_pair(GROUPS - 1, (GROUPS - 1) & 1))


def _gather_add(table, cidx):
    mesh = plsc.VectorSubcoreMesh(core_axis_name="c", subcore_axis_name="s")
    buf = pltpu.VMEM((GRP_EDGES, D_EDGE), jnp.float32)
    cbuf = pltpu.VMEM((D_EDGE, GRP_EDGES), jnp.float32)
    f = pl.kernel(
        _gather_body,
        out_type=jax.ShapeDtypeStruct((D_EDGE, E_PAD), jnp.float32),
        mesh=mesh,
        scratch_types=[
            pltpu.VMEM((2, EDGES_PER_TILE), jnp.int32),
            buf, buf, buf, buf, cbuf, cbuf,
            pltpu.SemaphoreType.DMA,
            pltpu.SemaphoreType.DMA,
        ],
        compiler_params=pltpu.CompilerParams(use_tc_tiling_on_sc=False,
                                             needs_layout_passes=False),
    )
    return f(table, cidx)


def _dense_body(st_ref, et_ref, w1t_ref, w2t_ref, pr_ref, o_ref):
    e = et_ref[...]
    x = (st_ref[...]
         + jnp.dot(w1t_ref[...], e, preferred_element_type=jnp.float32)
         + pr_ref[:, 0:1])
    h1 = x * (1.0 / (1.0 + jnp.exp(-x)))
    y = jnp.dot(w2t_ref[...], h1, preferred_element_type=jnp.float32) + pr_ref[:, 1:2]
    h2 = y * (1.0 / (1.0 + jnp.exp(-y)))
    z = e + h2
    m = jnp.mean(z, axis=0, keepdims=True)
    v = jnp.mean(z * z, axis=0, keepdims=True) - m * m
    o_ref[...] = (z - m) * lax.rsqrt(v + 1e-5) * pr_ref[:, 2:3] + pr_ref[:, 3:4]


def _dense(st, et, w1t, w2t, params_t):
    bc = 32000
    full = lambda t: (0, 0)
    col = lambda t: (0, t)
    return pl.pallas_call(
        _dense_body,
        grid=(N_EDGES // bc,),
        in_specs=[
            pl.BlockSpec((D_EDGE, bc), col),
            pl.BlockSpec((D_EDGE, bc), col),
            pl.BlockSpec((D_EDGE, D_EDGE), full),
            pl.BlockSpec((D_EDGE, D_EDGE), full),
            pl.BlockSpec((D_EDGE, 8), full),
        ],
        out_specs=pl.BlockSpec((D_EDGE, bc), col),
        out_shape=jax.ShapeDtypeStruct((D_EDGE, N_EDGES), jnp.float32),
    )(st, et, w1t, w2t, params_t)


def kernel(node_scalars, edge_index, edge_feats, W1, b1, W2, b2, gamma, beta):
    wa = W1[:D_NODE]
    wb = W1[D_NODE:2 * D_NODE]
    we = W1[2 * D_NODE:]

    table = _precompute_table(node_scalars, wa, wb)

    pad = E_PAD - N_EDGES
    src = jnp.pad(edge_index[0].astype(jnp.int32), (0, pad))
    dst = jnp.pad(edge_index[1].astype(jnp.int32), (0, pad)) + N_NODES
    cidx = jnp.stack([src.reshape(NW, EDGES_PER_TILE),
                      dst.reshape(NW, EDGES_PER_TILE)], axis=1)

    st = _gather_add(table, cidx)

    params_t = jnp.stack(
        [b1, b2, gamma, beta] + [jnp.zeros_like(b1)] * 4, axis=1)
    out_t = _dense(st, edge_feats.T, we.T, W2.T, params_t)
    return out_t.T


# speedup vs baseline: 1.0032x; 1.0032x over previous
import jax
import jax.numpy as jnp
from jax import lax
from jax.experimental import pallas as pl
from jax.experimental.pallas import tpu as pltpu
from jax.experimental.pallas import tpu_sc as plsc

N_NODES = 10000
N_EDGES = 320000
D_NODE = 128
D_EDGE = 16

NC = 2
NS = 16
NW = NC * NS
GRP_EDGES = 1024
GROUPS = 10
EDGES_PER_TILE = GRP_EDGES * GROUPS
E_PAD = NW * EDGES_PER_TILE


def _pq_body(ns_ref, wa_ref, wb_ref, t_ref):
    half = pl.num_programs(0) // 2
    t = pl.program_id(0)
    w = jnp.where(t < half, wa_ref[...], wb_ref[...])
    t_ref[...] = jnp.dot(ns_ref[...], w, preferred_element_type=jnp.float32)


def _precompute_table(ns, wa, wb):
    br = 2000
    nb = N_NODES // br
    return pl.pallas_call(
        _pq_body,
        grid=(2 * nb,),
        in_specs=[
            pl.BlockSpec((br, D_NODE), lambda t: (t % (N_NODES // 2000), 0)),
            pl.BlockSpec((D_NODE, D_EDGE), lambda t: (0, 0)),
            pl.BlockSpec((D_NODE, D_EDGE), lambda t: (0, 0)),
        ],
        out_specs=pl.BlockSpec((br, D_EDGE), lambda t: (t, 0)),
        out_shape=jax.ShapeDtypeStruct((2 * N_NODES, D_EDGE), jnp.float32),
    )(ns, wa, wb)


def _gather_body(t_hbm, cidx_hbm, st_hbm,
                 idx_v, a0, a1, b0, b1, c0, c1, gsem, wsem):
    wid = lax.axis_index("s") * NC + lax.axis_index("c")
    pltpu.sync_copy(cidx_hbm.at[wid], idx_v)
    base = wid * EDGES_PER_TILE
    ab = ((a0, b0), (a1, b1))
    cb = (c0, c1)
    iota16 = lax.iota(jnp.int32, 16)

    def g_pairs(g, par):
        sl = pl.ds(g * GRP_EDGES, GRP_EDGES)
        return ((t_hbm.at[idx_v.at[0, sl]], ab[par][0], gsem),
                (t_hbm.at[idx_v.at[1, sl]], ab[par][1], gsem))

    def w_pair(g, par):
        e0 = base + g * GRP_EDGES
        return ((cb[par], st_hbm.at[:, pl.ds(e0, GRP_EDGES)], wsem),)

    def fire(pairs):
        for s, d, sem in pairs:
            pltpu.async_copy(s, d, sem)

    def drain(pairs):
        for s, d, sem in pairs:
            pltpu.make_async_copy(s, d, sem).wait()

    fire(g_pairs(0, 0))
    for g in range(GROUPS):
        par = g & 1
        if g + 1 < GROUPS:
            fire(g_pairs(g + 1, 1 - par))
        drain(g_pairs(g, par))
        if g >= 2:
            drain(w_pair(g - 2, par))
        a, b = ab[par]
        c = cb[par]

        @pl.loop(0, GRP_EDGES // 16)
        def _addt(eb):
            rows = eb * 16 + iota16
            for j in range(D_EDGE):
                col = jnp.full((16,), j, jnp.int32)
                va = plsc.load_gather(a, [rows, col])
                vb = plsc.load_gather(b, [rows, col])
                c[j, pl.ds(eb * 16, 16)] = va + vb

        fire(w_pair(g, par))
    drain(w_pair(GROUPS - 2, (GROUPS - 2) & 1))
    drain(w_pair(GROUPS - 1, (GROUPS - 1) & 1))


def _gather_add(table, cidx):
    mesh = plsc.VectorSubcoreMesh(core_axis_name="c", subcore_axis_name="s")
    buf = pltpu.VMEM((GRP_EDGES, D_EDGE), jnp.float32)
    cbuf = pltpu.VMEM((D_EDGE, GRP_EDGES), jnp.float32)
    f = pl.kernel(
        _gather_body,
        out_type=jax.ShapeDtypeStruct((D_EDGE, E_PAD), jnp.float32),
        mesh=mesh,
        scratch_types=[
            pltpu.VMEM((2, EDGES_PER_TILE), jnp.int32),
            buf, buf, buf, buf, cbuf, cbuf,
            pltpu.SemaphoreType.DMA,
            pltpu.SemaphoreType.DMA,
        ],
        compiler_params=pltpu.CompilerParams(use_tc_tiling_on_sc=False,
                                             needs_layout_passes=False),
    )
    return f(table, cidx)


def _dense_body(st_ref, et_ref, w1t_ref, w2t_ref, pr_ref, o_ref):
    e = et_ref[...]
    x = (st_ref[...]
         + jnp.dot(w1t_ref[...], e, preferred_element_type=jnp.float32)
         + pr_ref[:, 0:1])
    h1 = x * (1.0 / (1.0 + jnp.exp(-x)))
    y = jnp.dot(w2t_ref[...], h1, preferred_element_type=jnp.float32) + pr_ref[:, 1:2]
    h2 = y * (1.0 / (1.0 + jnp.exp(-y)))
    z = e + h2
    m = jnp.mean(z, axis=0, keepdims=True)
    v = jnp.mean(z * z, axis=0, keepdims=True) - m * m
    o_ref[...] = (z - m) * lax.rsqrt(v + 1e-5) * pr_ref[:, 2:3] + pr_ref[:, 3:4]


def _dense(st, et, w1t, w2t, params_t):
    bc = 32000
    full = lambda t: (0, 0)
    col = lambda t: (0, t)
    return pl.pallas_call(
        _dense_body,
        grid=(N_EDGES // bc,),
        in_specs=[
            pl.BlockSpec((D_EDGE, bc), col),
            pl.BlockSpec((D_EDGE, bc), col),
            pl.BlockSpec((D_EDGE, D_EDGE), full),
            pl.BlockSpec((D_EDGE, D_EDGE), full),
            pl.BlockSpec((D_EDGE, 8), full),
        ],
        out_specs=pl.BlockSpec((D_EDGE, bc), col),
        out_shape=jax.ShapeDtypeStruct((D_EDGE, N_EDGES), jnp.float32),
    )(st, et, w1t, w2t, params_t)


def kernel(node_scalars, edge_index, edge_feats, W1, b1, W2, b2, gamma, beta):
    wa = W1[:D_NODE]
    wb = W1[D_NODE:2 * D_NODE]
    we = W1[2 * D_NODE:]

    table = _precompute_table(node_scalars, wa, wb)

    pad = E_PAD - N_EDGES
    src = jnp.pad(edge_index[0].astype(jnp.int32), (0, pad))
    dst = jnp.pad(edge_index[1].astype(jnp.int32), (0, pad)) + N_NODES
    cidx = jnp.stack([src.reshape(NW, EDGES_PER_TILE),
                      dst.reshape(NW, EDGES_PER_TILE)], axis=1)

    st = _gather_add(table, cidx)

    params_t = jnp.stack(
        [b1, b2, gamma, beta] + [jnp.zeros_like(b1)] * 4, axis=1)
    out_t = _dense(st, edge_feats.T, we.T, W2.T, params_t)
    return out_t.T
